# Initial kernel scaffold; baseline (speedup 1.0000x reference)
import jax
import jax.numpy as jnp
from jax.experimental import pallas as pl


def kernel(x_lead, x_pos_masked, mask_ratio, score_emb_w, Wq, Wk):
    raise NotImplementedError("write your pallas kernel here")


# speedup vs baseline: 7.5568x; 7.5568x over previous
import functools

import jax
import jax.numpy as jnp
from jax import lax
from jax.experimental import pallas as pl
from jax.experimental.pallas import tpu as pltpu
from jax.experimental.pallas import tpu_sc as plsc

B, L, D, H = 4, 2048, 1024, 16
DH = D // H
KEEP = L // 2


def _score_ids_kernel(x_ref, se_ref, wq_ref, wk_ref, ids_ref, flat_ref):
    b = pl.program_id(0)

    se_bf = se_ref[...].astype(jnp.bfloat16)
    wq_bf = wq_ref[...].astype(jnp.bfloat16)
    q_f32 = lax.dot_general(se_bf, wq_bf, (((1,), (1,)), ((), ())),
                            preferred_element_type=jnp.float32)
    q_bf = q_f32.astype(jnp.bfloat16)

    x_bf = x_ref[0].astype(jnp.bfloat16)
    wk_bf = wk_ref[...].astype(jnp.bfloat16)
    kt_f32 = lax.dot_general(wk_bf, x_bf, (((1,), (1,)), ((), ())),
                             preferred_element_type=jnp.float32)
    kt_bf = kt_f32.astype(jnp.bfloat16)

    col = lax.broadcasted_iota(jnp.int32, (H, D), 1) // DH
    hh = lax.broadcasted_iota(jnp.int32, (H, D), 0)
    ind = (jnp.int32(1) - jnp.minimum(jnp.abs(col - hh), jnp.int32(1)))
    qmat = jnp.broadcast_to(q_bf, (H, D)) * ind.astype(jnp.bfloat16)
    dots = lax.dot_general(qmat, kt_bf, (((1,), (0,)), ((), ())),
                           preferred_element_type=jnp.float32)

    t = dots * jnp.float32(0.125)
    m = jnp.max(t, axis=1, keepdims=True)
    e = jnp.exp(t - m)
    z = jnp.sum(e, axis=1, keepdims=True)
    p = e / z
    s = jnp.sum(p, axis=0, keepdims=True) * jnp.float32(0.0625)

    s_t = jnp.transpose(s)
    j_row = lax.broadcasted_iota(jnp.int32, (1, L), 1)
    rank = jnp.zeros((1, L), jnp.int32)
    CH = 256
    for c in range(L // CH):
        si = s_t[c * CH:(c + 1) * CH, :]
        ii = lax.broadcasted_iota(jnp.int32, (CH, 1), 0) + c * CH
        gt = si > s
        eq = (si == s) & (ii < j_row)
        contrib = jnp.where(gt | eq, jnp.int32(1), jnp.int32(0))
        rank = rank + jnp.sum(contrib, axis=0, keepdims=True)

    rank_t = jnp.transpose(rank)
    r_row = lax.broadcasted_iota(jnp.int32, (1, KEEP), 1)
    ids = jnp.zeros((1, KEEP), jnp.int32)
    for c in range(L // CH):
        rk = rank_t[c * CH:(c + 1) * CH, :]
        jj = lax.broadcasted_iota(jnp.int32, (CH, 1), 0) + c * CH
        hit = rk == r_row
        ids = ids + jnp.sum(jnp.where(hit, jj, jnp.int32(0)),
                            axis=0, keepdims=True)

    ids_ref[...] = ids.reshape(1, 1, KEEP)
    flat_ref[...] = (ids + b * L).reshape(1, 1, KEEP)


def _score_ids(x_lead, score_emb_w, Wq, Wk):
    return pl.pallas_call(
        _score_ids_kernel,
        grid=(B,),
        in_specs=[
            pl.BlockSpec((1, L, D), lambda b: (b, 0, 0)),
            pl.BlockSpec((1, D), lambda b: (0, 0)),
            pl.BlockSpec((D, D), lambda b: (0, 0)),
            pl.BlockSpec((D, D), lambda b: (0, 0)),
        ],
        out_specs=[
            pl.BlockSpec((1, 1, KEEP), lambda b: (b, 0, 0)),
            pl.BlockSpec((1, 1, KEEP), lambda b: (b, 0, 0)),
        ],
        out_shape=[
            jax.ShapeDtypeStruct((B, 1, KEEP), jnp.int32),
            jax.ShapeDtypeStruct((B, 1, KEEP), jnp.int32),
        ],
    )(x_lead, score_emb_w, Wq, Wk)


def _make_sc_gather():
    info = plsc.get_sparse_core_info()
    nw = info.num_cores * info.num_subcores
    rows_per_w = (B * KEEP) // nw
    CHUNK = 16
    nchunks = rows_per_w // CHUNK
    mesh = plsc.VectorSubcoreMesh(core_axis_name="c", subcore_axis_name="s")

    @functools.partial(
        pl.kernel, mesh=mesh,
        out_type=[
            jax.ShapeDtypeStruct((B * KEEP, D), jnp.float32),
            jax.ShapeDtypeStruct((B * KEEP, D), jnp.float32),
        ],
        scratch_types=[
            pltpu.VMEM((rows_per_w,), jnp.int32),
            pltpu.VMEM((CHUNK, D), jnp.float32),
            pltpu.VMEM((CHUNK, D), jnp.float32),
            pltpu.SemaphoreType.DMA,
            pltpu.SemaphoreType.DMA,
        ],
    )
    def sc_gather(x_hbm, xp_hbm, idx_hbm, out_x, out_xp,
                  idx_v, rows_a, rows_b, sem_a, sem_b):
        wid = lax.axis_index("s") * info.num_cores + lax.axis_index("c")
        base = wid * rows_per_w
        pltpu.sync_copy(idx_hbm.at[pl.ds(base, rows_per_w)], idx_v)
        for c in range(nchunks):
            idx_c = idx_v.at[pl.ds(c * CHUNK, CHUNK)]
            cp_a = pltpu.make_async_copy(x_hbm.at[idx_c], rows_a, sem_a)
            cp_a.start()
            cp_b = pltpu.make_async_copy(xp_hbm.at[idx_c], rows_b, sem_b)
            cp_b.start()
            cp_a.wait()
            pltpu.sync_copy(rows_a, out_x.at[pl.ds(base + c * CHUNK, CHUNK)])
            cp_b.wait()
            pltpu.sync_copy(rows_b, out_xp.at[pl.ds(base + c * CHUNK, CHUNK)])

    return sc_gather


def kernel(x_lead, x_pos_masked, mask_ratio, score_emb_w, Wq, Wk):
    del mask_ratio
    ids, flat_ids = _score_ids(x_lead, score_emb_w, Wq, Wk)
    gather = _make_sc_gather()
    xsel, xpsel = gather(x_lead.reshape(B * L, D),
                         x_pos_masked.reshape(B * L, D),
                         flat_ids.reshape(B * KEEP))
    return (xsel.reshape(B, KEEP, D), xpsel.reshape(B, KEEP, D),
            ids.reshape(B, KEEP))


# speedup vs baseline: 7.9205x; 1.0481x over previous
import functools

import jax
import jax.numpy as jnp
from jax import lax
from jax.experimental import pallas as pl
from jax.experimental.pallas import tpu as pltpu
from jax.experimental.pallas import tpu_sc as plsc

B, L, D, H = 4, 2048, 1024, 16
DH = D // H
KEEP = L // 2


def _score_ids_kernel(x_ref, se_ref, wq_ref, wk_ref, ids_ref, flat_ref):
    b = pl.program_id(0)

    se_bf = se_ref[...].astype(jnp.bfloat16)
    wq_bf = wq_ref[...].astype(jnp.bfloat16)
    q_f32 = lax.dot_general(se_bf, wq_bf, (((1,), (1,)), ((), ())),
                            preferred_element_type=jnp.float32)
    q_bf = q_f32.astype(jnp.bfloat16)

    x_bf = x_ref[0].astype(jnp.bfloat16)
    wk_bf = wk_ref[...].astype(jnp.bfloat16)
    kt_f32 = lax.dot_general(wk_bf, x_bf, (((1,), (1,)), ((), ())),
                             preferred_element_type=jnp.float32)
    kt_bf = kt_f32.astype(jnp.bfloat16)

    col = lax.broadcasted_iota(jnp.int32, (H, D), 1) // DH
    hh = lax.broadcasted_iota(jnp.int32, (H, D), 0)
    ind = (jnp.int32(1) - jnp.minimum(jnp.abs(col - hh), jnp.int32(1)))
    qmat = jnp.broadcast_to(q_bf, (H, D)) * ind.astype(jnp.bfloat16)
    dots = lax.dot_general(qmat, kt_bf, (((1,), (0,)), ((), ())),
                           preferred_element_type=jnp.float32)

    t = dots * jnp.float32(0.125)
    m = jnp.max(t, axis=1, keepdims=True)
    e = jnp.exp(t - m)
    z = jnp.sum(e, axis=1, keepdims=True)
    p = e / z
    s = jnp.sum(p, axis=0, keepdims=True) * jnp.float32(0.0625)

    s_t = jnp.transpose(s)
    j_row = lax.broadcasted_iota(jnp.int32, (1, L), 1)
    rank = jnp.zeros((1, L), jnp.int32)
    CH = 256
    for c in range(L // CH):
        si = s_t[c * CH:(c + 1) * CH, :]
        ii = lax.broadcasted_iota(jnp.int32, (CH, 1), 0) + c * CH
        gt = si > s
        eq = (si == s) & (ii < j_row)
        contrib = jnp.where(gt | eq, jnp.int32(1), jnp.int32(0))
        rank = rank + jnp.sum(contrib, axis=0, keepdims=True)

    rank_t = jnp.transpose(rank)
    r_row = lax.broadcasted_iota(jnp.int32, (1, KEEP), 1)
    ids = jnp.zeros((1, KEEP), jnp.int32)
    for c in range(L // CH):
        rk = rank_t[c * CH:(c + 1) * CH, :]
        jj = lax.broadcasted_iota(jnp.int32, (CH, 1), 0) + c * CH
        hit = rk == r_row
        ids = ids + jnp.sum(jnp.where(hit, jj, jnp.int32(0)),
                            axis=0, keepdims=True)

    ids_ref[...] = ids.reshape(1, 1, KEEP)
    flat_ref[...] = (ids + b * L).reshape(1, 1, KEEP)


def _score_ids(x_lead, score_emb_w, Wq, Wk):
    return pl.pallas_call(
        _score_ids_kernel,
        grid=(B,),
        in_specs=[
            pl.BlockSpec((1, L, D), lambda b: (b, 0, 0)),
            pl.BlockSpec((1, D), lambda b: (0, 0)),
            pl.BlockSpec((D, D), lambda b: (0, 0)),
            pl.BlockSpec((D, D), lambda b: (0, 0)),
        ],
        out_specs=[
            pl.BlockSpec((1, 1, KEEP), lambda b: (b, 0, 0)),
            pl.BlockSpec((1, 1, KEEP), lambda b: (b, 0, 0)),
        ],
        out_shape=[
            jax.ShapeDtypeStruct((B, 1, KEEP), jnp.int32),
            jax.ShapeDtypeStruct((B, 1, KEEP), jnp.int32),
        ],
    )(x_lead, score_emb_w, Wq, Wk)


def _make_sc_gather():
    info = plsc.get_sparse_core_info()
    nw = info.num_cores * info.num_subcores
    rows_per_w = (B * KEEP) // nw
    CHUNK = 16
    nchunks = rows_per_w // CHUNK
    mesh = plsc.VectorSubcoreMesh(core_axis_name="c", subcore_axis_name="s")

    @functools.partial(
        pl.kernel, mesh=mesh,
        out_type=[
            jax.ShapeDtypeStruct((B * KEEP, D), jnp.float32),
            jax.ShapeDtypeStruct((B * KEEP, D), jnp.float32),
        ],
        scratch_types=[
            pltpu.VMEM((rows_per_w,), jnp.int32),
            pltpu.VMEM((2, CHUNK, D), jnp.float32),
            pltpu.VMEM((2, CHUNK, D), jnp.float32),
        ] + [pltpu.SemaphoreType.DMA] * 8,
    )
    def sc_gather(x_hbm, xp_hbm, idx_hbm, out_x, out_xp,
                  idx_v, rows_x, rows_p,
                  gx0, gx1, gp0, gp1, ox0, ox1, op0, op1):
        wid = lax.axis_index("s") * info.num_cores + lax.axis_index("c")
        base = wid * rows_per_w
        pltpu.sync_copy(idx_hbm.at[pl.ds(base, rows_per_w)], idx_v)
        g_sem = {("x", 0): gx0, ("x", 1): gx1, ("p", 0): gp0, ("p", 1): gp1}
        o_sem = {("x", 0): ox0, ("x", 1): ox1, ("p", 0): op0, ("p", 1): op1}
        gathers, outs = {}, {}

        def start_gather(c):
            b = c % 2
            idx_c = idx_v.at[pl.ds(c * CHUNK, CHUNK)]
            gathers[(c, "x")] = pltpu.make_async_copy(
                x_hbm.at[idx_c], rows_x.at[b], g_sem[("x", b)])
            gathers[(c, "x")].start()
            gathers[(c, "p")] = pltpu.make_async_copy(
                xp_hbm.at[idx_c], rows_p.at[b], g_sem[("p", b)])
            gathers[(c, "p")].start()

        def drain_to_out(c):
            b = c % 2
            dst = pl.ds(base + c * CHUNK, CHUNK)
            gathers[(c, "x")].wait()
            outs[(c, "x")] = pltpu.make_async_copy(
                rows_x.at[b], out_x.at[dst], o_sem[("x", b)])
            outs[(c, "x")].start()
            gathers[(c, "p")].wait()
            outs[(c, "p")] = pltpu.make_async_copy(
                rows_p.at[b], out_xp.at[dst], o_sem[("p", b)])
            outs[(c, "p")].start()

        for c in range(nchunks):
            if c >= 2:
                outs[(c - 2, "x")].wait()
                outs[(c - 2, "p")].wait()
            start_gather(c)
            if c >= 1:
                drain_to_out(c - 1)
        drain_to_out(nchunks - 1)
        outs[(nchunks - 2, "x")].wait()
        outs[(nchunks - 2, "p")].wait()
        outs[(nchunks - 1, "x")].wait()
        outs[(nchunks - 1, "p")].wait()

    return sc_gather


def kernel(x_lead, x_pos_masked, mask_ratio, score_emb_w, Wq, Wk):
    del mask_ratio
    ids, flat_ids = _score_ids(x_lead, score_emb_w, Wq, Wk)
    gather = _make_sc_gather()
    xsel, xpsel = gather(x_lead.reshape(B * L, D),
                         x_pos_masked.reshape(B * L, D),
                         flat_ids.reshape(B * KEEP))
    return (xsel.reshape(B, KEEP, D), xpsel.reshape(B, KEEP, D),
            ids.reshape(B, KEEP))
